# baseline (speedup 1.0000x reference)
import numpy as np
import jax
import jax.numpy as jnp
from jax import lax
from jax.experimental import pallas as pl
from jax.experimental.pallas import tpu as pltpu
from jax.experimental.pallas import tpu_sc as plsc

_NC, _NS, _L = 2, 16, 16
_NW = _NC * _NS
_B = 4194304
_D = 5
_RPW = _B // _NW
_C = 2048
_NCHUNK = _RPW // _C

_BOUNDS = np.arange(0.0, 1.0, 1.0 / 22)[1:][1::2].astype(np.float32)


def _body(x_hbm, out_hbm, x_v, out_v):
    wid = lax.axis_index("s") * _NC + lax.axis_index("c")
    base = wid * _RPW
    iota5 = lax.iota(jnp.int32, _L) * _D

    def chunk(ci, carry):
        r0 = base + ci * _C
        pltpu.sync_copy(x_hbm.at[pl.ds(r0 * _D, _C * _D)], x_v)

        def step(i, carry2):
            flat0 = i * (_L * _D) + iota5
            acc = jnp.zeros((_L,), jnp.float32)
            for j in range(_D):
                xj = plsc.load_gather(x_v, [flat0 + j])
                acc = acc + jnp.where(xj > _BOUNDS[j], 1.0, 0.0)
                acc = acc + jnp.where(xj > _BOUNDS[j + _D], 1.0, 0.0)
            out_v[pl.ds(i * _L, _L)] = acc
            return carry2

        lax.fori_loop(0, _C // _L, step, 0)
        pltpu.sync_copy(out_v, out_hbm.at[pl.ds(r0, _C)])
        return carry

    lax.fori_loop(0, _NCHUNK, chunk, 0)


def kernel(x):
    f = pl.kernel(
        _body,
        out_type=jax.ShapeDtypeStruct((_B,), jnp.float32),
        mesh=plsc.VectorSubcoreMesh(
            core_axis_name="c", subcore_axis_name="s",
            num_cores=_NC, num_subcores=_NS,
        ),
        scratch_types=[
            pltpu.VMEM((_C * _D,), jnp.float32),
            pltpu.VMEM((_C,), jnp.float32),
        ],
        compiler_params=pltpu.CompilerParams(needs_layout_passes=False),
    )
    return f(x.reshape(_B * _D))


# speedup vs baseline: 22.8681x; 22.8681x over previous
import numpy as np
import jax
import jax.numpy as jnp
from jax.experimental import pallas as pl
from jax.experimental.pallas import tpu as pltpu

_B = 4194304
_D = 5
_RB = 65536
_BOUNDS = np.arange(0.0, 1.0, 1.0 / 22)[1:][1::2].astype(np.float32)


def _tc_body(xt_ref, o_ref):
    xt = xt_ref[...]
    js = jax.lax.broadcasted_iota(jnp.int32, (_D, 1), 0)
    tlo = jnp.full((_D, 1), float(_BOUNDS[0]), jnp.float32)
    thi = jnp.full((_D, 1), float(_BOUNDS[_D]), jnp.float32)
    for j in range(1, _D):
        tlo = jnp.where(js == j, float(_BOUNDS[j]), tlo)
        thi = jnp.where(js == j, float(_BOUNDS[j + _D]), thi)
    cnt = jnp.where(xt > tlo, 1.0, 0.0) + jnp.where(xt > thi, 1.0, 0.0)
    o_ref[...] = jnp.sum(cnt, axis=0)


def kernel(x):
    xt = x.T
    f = pl.pallas_call(
        _tc_body,
        grid=(_B // _RB,),
        in_specs=[pl.BlockSpec((_D, _RB), lambda i: (0, i))],
        out_specs=pl.BlockSpec((_RB,), lambda i: (i,)),
        out_shape=jax.ShapeDtypeStruct((_B,), jnp.float32),
    )
    return f(xt)
